# baseline (speedup 1.0000x reference)
import functools

import jax
import jax.numpy as jnp
from jax import lax
from jax.experimental import pallas as pl
from jax.experimental.pallas import tpu as pltpu
from jax.experimental.pallas import tpu_sc as plsc

B = 16384
C = 1000
ROWS = 2048
NB = B // ROWS

_INFO = plsc.get_sparse_core_info()
_NC, _NS, _L = _INFO.num_cores, _INFO.num_subcores, _INFO.num_lanes
_NW = _NC * _NS
_BPW = B // _NW
_GCHUNK = 128


def _sc_gather(lab_hbm, phi16_hbm, out_hbm, lab_v, idx_v, rows_v, sem):
    wid = lax.axis_index("s") * _NC + lax.axis_index("c")
    base = wid * _BPW
    pltpu.sync_copy(lab_hbm.at[pl.ds(base, _BPW)], lab_v)
    for g in range(_BPW // _L):
        i16 = base + g * _L + lax.iota(jnp.int32, _L)
        f = i16 * C + lab_v[pl.ds(g * _L, _L)]
        idx_v[pl.ds(g * _L, _L)] = lax.shift_right_logical(f, 4)
    for j in range(_BPW // _GCHUNK):
        pltpu.async_copy(
            phi16_hbm.at[idx_v.at[pl.ds(j * _GCHUNK, _GCHUNK)]],
            rows_v.at[pl.ds(j * _GCHUNK, _GCHUNK), :],
            sem,
        ).wait()
    pltpu.sync_copy(rows_v, out_hbm.at[pl.ds(base, _BPW), :])


_sc_gather_call = functools.partial(
    pl.kernel,
    mesh=plsc.VectorSubcoreMesh(core_axis_name="c", subcore_axis_name="s"),
    out_type=jax.ShapeDtypeStruct((B, _L), jnp.float32),
    scratch_types=[
        pltpu.VMEM((_BPW,), jnp.int32),
        pltpu.VMEM((_BPW,), jnp.int32),
        pltpu.VMEM((_BPW, _L), jnp.float32),
        pltpu.SemaphoreType.DMA,
    ],
    compiler_params=pltpu.CompilerParams(use_tc_tiling_on_sc=False),
)(_sc_gather)


def _k1_body(cos_ref, lab_ref, m0_ref, s0_ref, cl_ref):
    cos = cos_ref[...]
    lab = lab_ref[0, 0, :]
    col = lax.broadcasted_iota(jnp.int32, (ROWS, C), 1)
    mask = col == lab[:, None]
    m0 = jnp.max(cos, axis=1)
    cl = jnp.sum(jnp.where(mask, cos, 0.0), axis=1)
    s0 = jnp.sum(jnp.exp(cos - m0[:, None]), axis=1)
    m0_ref[0, 0, :] = m0
    s0_ref[0, 0, :] = s0
    cl_ref[0, 0, :] = cl


def _k2_body(m0_ref, s0_ref, cl_ref, lab_ref, rows_ref, out_ref):
    m0 = m0_ref[0, 0, :]
    s0 = s0_ref[0, 0, :]
    cl = cl_ref[0, 0, :]
    lab = lab_ref[0, 0, :]
    rows = rows_ref[...]
    i2 = pl.program_id(0) * ROWS + lax.broadcasted_iota(jnp.int32, (ROWS, _L), 0)
    lane = jnp.bitwise_and(i2 * C + lab[:, None], _L - 1)
    lane16 = lax.broadcasted_iota(jnp.int32, (ROWS, _L), 1)
    phil = jnp.sum(jnp.where(lane16 == lane, rows, 0.0), axis=1)
    m = jnp.maximum(m0, phil)
    s = s0 * jnp.exp(m0 - m) - jnp.exp(cl - m) + jnp.exp(phil - m)
    nll = m + jnp.log(s) - phil

    @pl.when(pl.program_id(0) == 0)
    def _():
        out_ref[...] = jnp.zeros((1, 1), jnp.float32)

    out_ref[...] += jnp.sum(nll).reshape(1, 1)


def kernel(cos_theta, phi_theta, labels):
    phi16 = phi_theta.reshape(B * C // _L, _L)
    rows = _sc_gather_call(labels, phi16)

    lab3 = labels.reshape(NB, 1, ROWS)
    stat_shape = jax.ShapeDtypeStruct((NB, 1, ROWS), jnp.float32)
    stat_spec = pl.BlockSpec((1, 1, ROWS), lambda i: (i, 0, 0))
    m0, s0, cl = pl.pallas_call(
        _k1_body,
        grid=(NB,),
        in_specs=[
            pl.BlockSpec((ROWS, C), lambda i: (i, 0)),
            pl.BlockSpec((1, 1, ROWS), lambda i: (i, 0, 0)),
        ],
        out_specs=[stat_spec, stat_spec, stat_spec],
        out_shape=[stat_shape, stat_shape, stat_shape],
    )(cos_theta, lab3)

    total = pl.pallas_call(
        _k2_body,
        grid=(NB,),
        in_specs=[
            stat_spec,
            stat_spec,
            stat_spec,
            pl.BlockSpec((1, 1, ROWS), lambda i: (i, 0, 0)),
            pl.BlockSpec((ROWS, _L), lambda i: (i, 0)),
        ],
        out_specs=pl.BlockSpec((1, 1), lambda i: (0, 0)),
        out_shape=jax.ShapeDtypeStruct((1, 1), jnp.float32),
    )(m0, s0, cl, lab3, rows)
    return total[0, 0] / B


# speedup vs baseline: 1.5109x; 1.5109x over previous
import jax
import jax.numpy as jnp
from jax import lax
from jax.experimental import pallas as pl

B = 16384
C = 1000
ROWS = 2048


def _body(cos_ref, phi_ref, lab_ref, out_ref):
    cos = cos_ref[...].astype(jnp.float32)
    phi = phi_ref[...].astype(jnp.float32)
    lab = lab_ref[0, 0, :]
    col = lax.broadcasted_iota(jnp.int32, (ROWS, C), 1)
    mask = col == lab[:, None]
    z = jnp.where(mask, phi, cos)
    phil = jnp.sum(jnp.where(mask, phi, 0.0), axis=1)
    m = jnp.max(z, axis=1)
    s = jnp.sum(jnp.exp(z - m[:, None]), axis=1)
    nll = m + jnp.log(s) - phil

    @pl.when(pl.program_id(0) == 0)
    def _():
        out_ref[...] = jnp.zeros((1, 1), jnp.float32)

    out_ref[...] += jnp.sum(nll).reshape(1, 1)


def kernel(cos_theta, phi_theta, labels):
    cos_h = cos_theta.astype(jnp.bfloat16)
    phi_h = phi_theta.astype(jnp.bfloat16)
    nb = B // ROWS
    lab3 = labels.reshape(nb, 1, ROWS)
    total = pl.pallas_call(
        _body,
        grid=(nb,),
        in_specs=[
            pl.BlockSpec((ROWS, C), lambda i: (i, 0)),
            pl.BlockSpec((ROWS, C), lambda i: (i, 0)),
            pl.BlockSpec((1, 1, ROWS), lambda i: (i, 0, 0)),
        ],
        out_specs=pl.BlockSpec((1, 1), lambda i: (0, 0)),
        out_shape=jax.ShapeDtypeStruct((1, 1), jnp.float32),
    )(cos_h, phi_h, lab3)
    return total[0, 0] / B


# speedup vs baseline: 1.7085x; 1.1308x over previous
import jax
import jax.numpy as jnp
from jax import lax
from jax.experimental import pallas as pl

B = 16384
C = 1000
ROWS = 2048


def _body(cos_ref, phi_ref, lab_ref, out_ref):
    cos = cos_ref[...]
    phi = phi_ref[...]
    lab = lab_ref[0, 0, :]
    col = lax.broadcasted_iota(jnp.int32, (ROWS, C), 1)
    mask = col == lab[:, None]
    z = jnp.where(mask, phi, cos)
    phil = jnp.sum(jnp.where(mask, phi, 0.0), axis=1)
    m = jnp.max(z, axis=1)
    s = jnp.sum(jnp.exp(z - m[:, None]), axis=1)
    nll = m + jnp.log(s) - phil

    @pl.when(pl.program_id(0) == 0)
    def _():
        out_ref[...] = jnp.zeros((1, 1), jnp.float32)

    out_ref[...] += jnp.sum(nll).reshape(1, 1)


def kernel(cos_theta, phi_theta, labels):
    nb = B // ROWS
    lab3 = labels.reshape(nb, 1, ROWS)
    total = pl.pallas_call(
        _body,
        grid=(nb,),
        in_specs=[
            pl.BlockSpec((ROWS, C), lambda i: (i, 0)),
            pl.BlockSpec((ROWS, C), lambda i: (i, 0)),
            pl.BlockSpec((1, 1, ROWS), lambda i: (i, 0, 0)),
        ],
        out_specs=pl.BlockSpec((1, 1), lambda i: (0, 0)),
        out_shape=jax.ShapeDtypeStruct((1, 1), jnp.float32),
    )(cos_theta, phi_theta, lab3)
    return total[0, 0] / B
